# Initial kernel scaffold; baseline (speedup 1.0000x reference)
import jax
import jax.numpy as jnp
from jax.experimental import pallas as pl


def kernel(idx, table):
    raise NotImplementedError("write your pallas kernel here")


# speedup vs baseline: 1.6841x; 1.6841x over previous
import functools

import jax
import jax.numpy as jnp
from jax import lax
from jax.experimental import pallas as pl
from jax.experimental.pallas import tpu as pltpu
from jax.experimental.pallas import tpu_sc as plsc

_B_TOTAL = 16384 * 50
_D = 64
_NC = 2
_NS = 16
_NW = _NC * _NS
_B_PER_W = _B_TOTAL // _NW
_CHUNK = 128
_N_CHUNKS = _B_PER_W // _CHUNK


def _make_gather():
  mesh = plsc.VectorSubcoreMesh(core_axis_name="c", subcore_axis_name="s")

  @functools.partial(
      pl.kernel,
      out_type=jax.ShapeDtypeStruct((_B_TOTAL, _D), jnp.float32),
      mesh=mesh,
      compiler_params=pltpu.CompilerParams(use_tc_tiling_on_sc=False),
      scratch_types=[
          pltpu.VMEM((_N_CHUNKS, _CHUNK), jnp.int32),
          pltpu.VMEM((2, _CHUNK, _D), jnp.float32),
          pltpu.SemaphoreType.DMA,
      ],
  )
  def gather_kernel(table_hbm, idx_hbm, out_hbm, idx_v, rows_v, sem):
    wid = lax.axis_index("s") * _NC + lax.axis_index("c")
    base = wid * _B_PER_W
    pltpu.sync_copy(idx_hbm.at[wid], idx_v)

    def chunk_body(j, carry):
      pltpu.async_copy(table_hbm.at[idx_v.at[j]], rows_v.at[0], sem).wait()
      pltpu.sync_copy(rows_v.at[0], out_hbm.at[pl.ds(base + j * _CHUNK, _CHUNK)])
      return carry

    lax.fori_loop(0, _N_CHUNKS, chunk_body, 0, unroll=False)

  return gather_kernel


_gather = _make_gather()


def kernel(idx, table):
  idx3 = idx.reshape(_NW, _N_CHUNKS, _CHUNK).astype(jnp.int32)
  out = _gather(table, idx3)
  return out.reshape(idx.shape + (_D,))


# speedup vs baseline: 1.9576x; 1.1623x over previous
import functools

import jax
import jax.numpy as jnp
from jax import lax
from jax.experimental import pallas as pl
from jax.experimental.pallas import tpu as pltpu
from jax.experimental.pallas import tpu_sc as plsc

_B_TOTAL = 16384 * 50
_D = 64
_NC = 2
_NS = 16
_NW = _NC * _NS
_B_PER_W = _B_TOTAL // _NW
_CHUNK = 128
_N_CHUNKS = _B_PER_W // _CHUNK


def _make_gather():
  mesh = plsc.VectorSubcoreMesh(core_axis_name="c", subcore_axis_name="s")

  @functools.partial(
      pl.kernel,
      out_type=jax.ShapeDtypeStruct((_B_TOTAL, _D), jnp.float32),
      mesh=mesh,
      compiler_params=pltpu.CompilerParams(use_tc_tiling_on_sc=False),
      scratch_types=[
          pltpu.VMEM((_N_CHUNKS, _CHUNK), jnp.int32),
          pltpu.VMEM((2, _CHUNK, _D), jnp.float32),
          pltpu.SemaphoreType.DMA,
          pltpu.SemaphoreType.DMA,
          pltpu.SemaphoreType.DMA,
          pltpu.SemaphoreType.DMA,
      ],
  )
  def gather_kernel(table_hbm, idx_hbm, out_hbm, idx_v, rows_v, gs0, gs1,
                    os0, os1):
    wid = lax.axis_index("s") * _NC + lax.axis_index("c")
    base = wid * _B_PER_W
    pltpu.sync_copy(idx_hbm.at[wid], idx_v)

    def gather_start(j, buf, gsem):
      pltpu.async_copy(table_hbm.at[idx_v.at[j]], rows_v.at[buf], gsem)

    def gather_wait(j, buf, gsem):
      pltpu.make_async_copy(
          table_hbm.at[idx_v.at[j]], rows_v.at[buf], gsem).wait()

    def out_start(j, buf, osem):
      pltpu.async_copy(
          rows_v.at[buf], out_hbm.at[pl.ds(base + j * _CHUNK, _CHUNK)], osem)

    def out_wait(j, buf, osem):
      pltpu.make_async_copy(
          rows_v.at[buf], out_hbm.at[pl.ds(base + j * _CHUNK, _CHUNK)],
          osem).wait()

    gather_start(0, 0, gs0)
    n_pairs = _N_CHUNKS // 2

    def step(i, carry):
      j0 = 2 * i
      gather_wait(j0, 0, gs0)

      @pl.when(i > 0)
      def _():
        out_wait(j0 - 1, 1, os1)

      gather_start(j0 + 1, 1, gs1)
      out_start(j0, 0, os0)

      gather_wait(j0 + 1, 1, gs1)
      out_wait(j0, 0, os0)

      @pl.when(i + 1 < n_pairs)
      def _():
        gather_start(j0 + 2, 0, gs0)

      out_start(j0 + 1, 1, os1)
      return carry

    lax.fori_loop(0, n_pairs, step, 0, unroll=False)
    out_wait(_N_CHUNKS - 1, 1, os1)

  return gather_kernel


_gather = _make_gather()


def kernel(idx, table):
  idx3 = idx.reshape(_NW, _N_CHUNKS, _CHUNK).astype(jnp.int32)
  out = _gather(table, idx3)
  return out.reshape(idx.shape + (_D,))
